# baseline (speedup 1.0000x reference)
import functools

import jax
import jax.numpy as jnp
from jax import lax
from jax.experimental import pallas as pl
from jax.experimental.pallas import tpu as pltpu
from jax.experimental.pallas import tpu_sc as plsc

K_TOK = 8192
K_CODE = 8192
E = 256
MB = 512
NB = 2048
GM = K_TOK // MB
GN = K_CODE // NB
N_ELEMS = K_TOK * E


def _dist_body(z_ref, w_ref, zsq_ref, wsq_ref, idx_ref, loss_ref,
               rmin_ref, ridx_ref, sel_ref):
    m = pl.program_id(0)
    n = pl.program_id(1)
    zb = z_ref[...]
    wb = w_ref[...]
    a = zsq_ref[...]
    b = wsq_ref[...]
    mm2 = lax.dot_general(zb, wb, (((1,), (1,)), ((), ())),
                          preferred_element_type=jnp.float32)
    d = (a + b) - mm2
    bmin = jnp.min(d, axis=1, keepdims=True)
    col = lax.broadcasted_iota(jnp.int32, (1, NB), 1).astype(jnp.float32)
    bidx = jnp.min(jnp.where(d == bmin, col, jnp.float32(jnp.inf)),
                   axis=1, keepdims=True).astype(jnp.int32)

    @pl.when(n == 0)
    def _init():
        rmin_ref[...] = bmin.astype(jnp.bfloat16).astype(jnp.float32)
        ridx_ref[...] = bidx
        sel_ref[...] = bmin

    @pl.when(n > 0)
    def _update():
        prev = rmin_ref[...]
        better = bmin < prev
        rmin_ref[...] = (jnp.where(better, bmin, prev)
                         .astype(jnp.bfloat16).astype(jnp.float32))
        ridx_ref[...] = jnp.where(better, bidx + n * NB, ridx_ref[...])
        sel_ref[...] = jnp.where(better, bmin, sel_ref[...])

    @pl.when(n == GN - 1)
    def _finish():
        idx_ref[...] = ridx_ref[...]
        s = jnp.sum(sel_ref[...])

        @pl.when(m == 0)
        def _():
            loss_ref[0] = s

        @pl.when(m > 0)
        def _():
            loss_ref[0] += s


def _distance_argmin(z_flat, W, zsq, wsq):
    return pl.pallas_call(
        _dist_body,
        grid=(GM, GN),
        in_specs=[
            pl.BlockSpec((MB, E), lambda m, n: (m, 0)),
            pl.BlockSpec((NB, E), lambda m, n: (n, 0)),
            pl.BlockSpec((MB, 1), lambda m, n: (m, 0)),
            pl.BlockSpec((1, NB), lambda m, n: (0, n)),
        ],
        out_specs=[
            pl.BlockSpec((MB, 1), lambda m, n: (m, 0)),
            pl.BlockSpec(memory_space=pltpu.SMEM, block_shape=(1,),
                         index_map=lambda m, n: (0,)),
        ],
        out_shape=[
            jax.ShapeDtypeStruct((K_TOK, 1), jnp.int32),
            jax.ShapeDtypeStruct((1,), jnp.float32),
        ],
        scratch_shapes=[
            pltpu.VMEM((MB, 1), jnp.float32),
            pltpu.VMEM((MB, 1), jnp.int32),
            pltpu.VMEM((MB, 1), jnp.float32),
        ],
    )(z_flat, W, zsq, wsq)


def _sc_gather(table, idx):
    try:
        info = plsc.get_sparse_core_info()
        nc, ns = info.num_cores, info.num_subcores
    except Exception:
        nc, ns = 2, 16
    nw = nc * ns
    b_per_w = K_TOK // nw
    ch = 128
    n_ch = b_per_w // ch
    mesh = plsc.VectorSubcoreMesh(core_axis_name="c", subcore_axis_name="s")

    @functools.partial(
        pl.kernel, mesh=mesh,
        out_type=jax.ShapeDtypeStruct((K_TOK, E), jnp.float32),
        scratch_types=[
            pltpu.VMEM((ch,), jnp.int32),
            pltpu.VMEM((ch, E), jnp.float32),
            pltpu.SemaphoreType.DMA,
        ],
    )
    def _gather(table_hbm, idx_hbm, out_hbm, idx_v, rows_v, sem):
        wid = lax.axis_index("s") * nc + lax.axis_index("c")
        base = wid * b_per_w
        for j in range(n_ch):
            off = base + j * ch
            pltpu.sync_copy(idx_hbm.at[pl.ds(off, ch)], idx_v)
            pltpu.async_copy(table_hbm.at[idx_v], rows_v, sem).wait()
            pltpu.sync_copy(rows_v, out_hbm.at[pl.ds(off, ch)])

    return _gather(table, idx)


def kernel(z, W):
    zp = jnp.transpose(z, (0, 2, 3, 1))
    z_flat = zp.reshape(K_TOK, E)
    zsq = jnp.sum(zp ** 2, axis=3).reshape(K_TOK, 1)
    wsq = jnp.sum(W ** 2, axis=1).reshape(1, K_CODE)
    idx2d, loss_sum = _distance_argmin(z_flat * 2.0, W, zsq, wsq)
    idx = idx2d.reshape(K_TOK)
    table = W.astype(jnp.bfloat16).astype(jnp.float32)
    vq_rows = _sc_gather(table, idx)
    loss = loss_sum[0] * jnp.float32(1.25 / N_ELEMS)
    vq_tok = z_flat + (vq_rows - z_flat)
    vq_out = jnp.transpose(vq_tok.reshape(8, 32, 32, E), (0, 3, 1, 2))
    return (loss, vq_out)


# speedup vs baseline: 1.0167x; 1.0167x over previous
import functools

import jax
import jax.numpy as jnp
from jax import lax
from jax.experimental import pallas as pl
from jax.experimental.pallas import tpu as pltpu
from jax.experimental.pallas import tpu_sc as plsc

K_TOK = 8192
K_CODE = 8192
E = 256
MB = 512
NB = 2048
GM = K_TOK // MB
GN = K_CODE // NB
N_ELEMS = K_TOK * E


def _dist_body(z_ref, w_ref, zsq_ref, wsq_ref, idx_ref, loss_ref,
               rmin_ref, ridx_ref, sel_ref):
    m = pl.program_id(0)
    n = pl.program_id(1)
    zb = z_ref[...]
    wb = w_ref[...]
    a = zsq_ref[...]
    b = wsq_ref[...]
    mm2 = lax.dot_general(zb, wb, (((1,), (1,)), ((), ())),
                          preferred_element_type=jnp.float32)
    d = (a + b) - mm2
    bmin = jnp.min(d, axis=1, keepdims=True)
    col = lax.broadcasted_iota(jnp.int32, (1, NB), 1).astype(jnp.float32)
    bidx = jnp.min(jnp.where(d == bmin, col, jnp.float32(jnp.inf)),
                   axis=1, keepdims=True).astype(jnp.int32)

    @pl.when(n == 0)
    def _init():
        rmin_ref[...] = bmin.astype(jnp.bfloat16).astype(jnp.float32)
        ridx_ref[...] = bidx
        sel_ref[...] = bmin

    @pl.when(n > 0)
    def _update():
        prev = rmin_ref[...]
        better = bmin < prev
        rmin_ref[...] = (jnp.where(better, bmin, prev)
                         .astype(jnp.bfloat16).astype(jnp.float32))
        ridx_ref[...] = jnp.where(better, bidx + n * NB, ridx_ref[...])
        sel_ref[...] = jnp.where(better, bmin, sel_ref[...])

    @pl.when(n == GN - 1)
    def _finish():
        idx_ref[...] = ridx_ref[...].reshape(MB)
        s = jnp.sum(sel_ref[...])

        @pl.when(m == 0)
        def _():
            loss_ref[0] = s

        @pl.when(m > 0)
        def _():
            loss_ref[0] += s


def _distance_argmin(z_flat, W, zsq, wsq):
    return pl.pallas_call(
        _dist_body,
        grid=(GM, GN),
        in_specs=[
            pl.BlockSpec((MB, E), lambda m, n: (m, 0)),
            pl.BlockSpec((NB, E), lambda m, n: (n, 0)),
            pl.BlockSpec((MB, 1), lambda m, n: (m, 0)),
            pl.BlockSpec((1, NB), lambda m, n: (0, n)),
        ],
        out_specs=[
            pl.BlockSpec((MB,), lambda m, n: (m,)),
            pl.BlockSpec(memory_space=pltpu.SMEM, block_shape=(1,),
                         index_map=lambda m, n: (0,)),
        ],
        out_shape=[
            jax.ShapeDtypeStruct((K_TOK,), jnp.int32),
            jax.ShapeDtypeStruct((1,), jnp.float32),
        ],
        scratch_shapes=[
            pltpu.VMEM((MB, 1), jnp.float32),
            pltpu.VMEM((MB, 1), jnp.int32),
            pltpu.VMEM((MB, 1), jnp.float32),
        ],
    )(z_flat, W, zsq, wsq)


def _sc_gather(table, idx):
    try:
        info = plsc.get_sparse_core_info()
        nc, ns = info.num_cores, info.num_subcores
    except Exception:
        nc, ns = 2, 16
    nw = nc * ns
    b_per_w = K_TOK // nw
    ch = 128
    n_ch = b_per_w // ch
    mesh = plsc.VectorSubcoreMesh(core_axis_name="c", subcore_axis_name="s")

    @functools.partial(
        pl.kernel, mesh=mesh,
        out_type=jax.ShapeDtypeStruct((K_TOK, E), jnp.float32),
        scratch_types=[
            pltpu.VMEM((ch,), jnp.int32),
            pltpu.VMEM((ch, E), jnp.float32),
            pltpu.SemaphoreType.DMA,
        ],
    )
    def _gather(table_hbm, idx_hbm, out_hbm, idx_v, rows_v, sem):
        wid = lax.axis_index("s") * nc + lax.axis_index("c")
        base = wid * b_per_w
        for j in range(n_ch):
            off = base + j * ch
            pltpu.sync_copy(idx_hbm.at[pl.ds(off, ch)], idx_v)
            pltpu.async_copy(table_hbm.at[idx_v], rows_v, sem).wait()
            pltpu.sync_copy(rows_v, out_hbm.at[pl.ds(off, ch)])

    return _gather(table, idx)


def kernel(z, W):
    zp = jnp.transpose(z, (0, 2, 3, 1))
    z_flat = zp.reshape(K_TOK, E)
    zsq = jnp.sum(zp ** 2, axis=3).reshape(K_TOK, 1)
    wsq = jnp.sum(W ** 2, axis=1).reshape(1, K_CODE)
    idx, loss_sum = _distance_argmin(z_flat * 2.0, W, zsq, wsq)
    table = W.astype(jnp.bfloat16).astype(jnp.float32)
    vq_rows = _sc_gather(table, idx)
    loss = loss_sum[0] * jnp.float32(1.25 / N_ELEMS)
    vq_tok = z_flat + (vq_rows - z_flat)
    vq_out = jnp.transpose(vq_tok.reshape(8, 32, 32, E), (0, 3, 1, 2))
    return (loss, vq_out)


# speedup vs baseline: 1.1052x; 1.0871x over previous
import functools

import jax
import jax.numpy as jnp
from jax import lax
from jax.experimental import pallas as pl
from jax.experimental.pallas import tpu as pltpu
from jax.experimental.pallas import tpu_sc as plsc

K_TOK = 8192
K_CODE = 8192
E = 256
MB = 1024
NB = 2048
GM = K_TOK // MB
GN = K_CODE // NB
N_ELEMS = K_TOK * E


def _dist_body(z_ref, w_ref, zsq_ref, wsq_ref, idx_ref, loss_ref,
               rmin_ref, ridx_ref, sel_ref):
    m = pl.program_id(0)
    n = pl.program_id(1)
    zb = z_ref[...]
    wb = w_ref[...]
    a = zsq_ref[...]
    b = wsq_ref[...]
    mm2 = lax.dot_general(zb, wb, (((1,), (1,)), ((), ())),
                          preferred_element_type=jnp.float32)
    d = (a + b) - mm2
    bmin = jnp.min(d, axis=1, keepdims=True)
    col = lax.broadcasted_iota(jnp.int32, (1, NB), 1).astype(jnp.float32)
    bidx = jnp.min(jnp.where(d == bmin, col, jnp.float32(jnp.inf)),
                   axis=1, keepdims=True).astype(jnp.int32)

    @pl.when(n == 0)
    def _init():
        rmin_ref[...] = bmin.astype(jnp.bfloat16).astype(jnp.float32)
        ridx_ref[...] = bidx
        sel_ref[...] = bmin

    @pl.when(n > 0)
    def _update():
        prev = rmin_ref[...]
        better = bmin < prev
        rmin_ref[...] = (jnp.where(better, bmin, prev)
                         .astype(jnp.bfloat16).astype(jnp.float32))
        ridx_ref[...] = jnp.where(better, bidx + n * NB, ridx_ref[...])
        sel_ref[...] = jnp.where(better, bmin, sel_ref[...])

    @pl.when(n == GN - 1)
    def _finish():
        idx_ref[...] = ridx_ref[...].reshape(MB)
        s = jnp.sum(sel_ref[...])

        @pl.when(m == 0)
        def _():
            loss_ref[0] = s

        @pl.when(m > 0)
        def _():
            loss_ref[0] += s


def _distance_argmin(z_flat, W, zsq, wsq):
    return pl.pallas_call(
        _dist_body,
        grid=(GM, GN),
        in_specs=[
            pl.BlockSpec((MB, E), lambda m, n: (m, 0)),
            pl.BlockSpec((NB, E), lambda m, n: (n, 0)),
            pl.BlockSpec((MB, 1), lambda m, n: (m, 0)),
            pl.BlockSpec((1, NB), lambda m, n: (0, n)),
        ],
        out_specs=[
            pl.BlockSpec((MB,), lambda m, n: (m,)),
            pl.BlockSpec(memory_space=pltpu.SMEM, block_shape=(1,),
                         index_map=lambda m, n: (0,)),
        ],
        out_shape=[
            jax.ShapeDtypeStruct((K_TOK,), jnp.int32),
            jax.ShapeDtypeStruct((1,), jnp.float32),
        ],
        scratch_shapes=[
            pltpu.VMEM((MB, 1), jnp.float32),
            pltpu.VMEM((MB, 1), jnp.int32),
            pltpu.VMEM((MB, 1), jnp.float32),
        ],
    )(z_flat, W, zsq, wsq)


def _sc_gather(table, idx):
    try:
        info = plsc.get_sparse_core_info()
        nc, ns = info.num_cores, info.num_subcores
    except Exception:
        nc, ns = 2, 16
    nw = nc * ns
    b_per_w = K_TOK // nw
    ch = 128
    n_ch = b_per_w // ch
    mesh = plsc.VectorSubcoreMesh(core_axis_name="c", subcore_axis_name="s")

    @functools.partial(
        pl.kernel, mesh=mesh,
        out_type=jax.ShapeDtypeStruct((K_TOK, E), jnp.float32),
        scratch_types=[
            pltpu.VMEM((ch,), jnp.int32),
            pltpu.VMEM((ch, E), jnp.float32),
            pltpu.SemaphoreType.DMA,
        ],
    )
    def _gather(table_hbm, idx_hbm, out_hbm, idx_v, rows_v, sem):
        wid = lax.axis_index("s") * nc + lax.axis_index("c")
        base = wid * b_per_w
        for j in range(n_ch):
            off = base + j * ch
            pltpu.sync_copy(idx_hbm.at[pl.ds(off, ch)], idx_v)
            pltpu.async_copy(table_hbm.at[idx_v], rows_v, sem).wait()
            pltpu.sync_copy(rows_v, out_hbm.at[pl.ds(off, ch)])

    return _gather(table, idx)


def kernel(z, W):
    zp = jnp.transpose(z, (0, 2, 3, 1))
    z_flat = zp.reshape(K_TOK, E)
    zsq = jnp.sum(zp ** 2, axis=3).reshape(K_TOK, 1)
    wsq = jnp.sum(W ** 2, axis=1).reshape(1, K_CODE)
    idx, loss_sum = _distance_argmin(z_flat * 2.0, W, zsq, wsq)
    table = W.astype(jnp.bfloat16).astype(jnp.float32)
    vq_rows = _sc_gather(table, idx)
    loss = loss_sum[0] * jnp.float32(1.25 / N_ELEMS)
    vq_tok = z_flat + (vq_rows - z_flat)
    vq_out = jnp.transpose(vq_tok.reshape(8, 32, 32, E), (0, 3, 1, 2))
    return (loss, vq_out)


# speedup vs baseline: 1.1540x; 1.0441x over previous
import functools

import jax
import jax.numpy as jnp
from jax import lax
from jax.experimental import pallas as pl
from jax.experimental.pallas import tpu as pltpu
from jax.experimental.pallas import tpu_sc as plsc

K_TOK = 8192
K_CODE = 8192
E = 256
MB = 2048
NB = 2048
GM = K_TOK // MB
GN = K_CODE // NB
N_ELEMS = K_TOK * E


def _dist_body(z_ref, w_ref, zsq_ref, wsq_ref, idx_ref, loss_ref,
               rmin_ref, ridx_ref, sel_ref):
    m = pl.program_id(0)
    n = pl.program_id(1)
    zb = z_ref[...]
    wb = w_ref[...]
    a = zsq_ref[...]
    b = wsq_ref[...]
    mm2 = lax.dot_general(zb, wb, (((1,), (1,)), ((), ())),
                          preferred_element_type=jnp.float32)
    d = (a + b) - mm2
    bmin = jnp.min(d, axis=1, keepdims=True)
    col = lax.broadcasted_iota(jnp.int32, (1, NB), 1).astype(jnp.float32)
    bidx = jnp.min(jnp.where(d == bmin, col, jnp.float32(jnp.inf)),
                   axis=1, keepdims=True).astype(jnp.int32)

    @pl.when(n == 0)
    def _init():
        rmin_ref[...] = bmin.astype(jnp.bfloat16).astype(jnp.float32)
        ridx_ref[...] = bidx
        sel_ref[...] = bmin

    @pl.when(n > 0)
    def _update():
        prev = rmin_ref[...]
        better = bmin < prev
        rmin_ref[...] = (jnp.where(better, bmin, prev)
                         .astype(jnp.bfloat16).astype(jnp.float32))
        ridx_ref[...] = jnp.where(better, bidx + n * NB, ridx_ref[...])
        sel_ref[...] = jnp.where(better, bmin, sel_ref[...])

    @pl.when(n == GN - 1)
    def _finish():
        idx_ref[...] = ridx_ref[...].reshape(MB)
        s = jnp.sum(sel_ref[...])

        @pl.when(m == 0)
        def _():
            loss_ref[0] = s

        @pl.when(m > 0)
        def _():
            loss_ref[0] += s


def _distance_argmin(z_flat, W, zsq, wsq):
    return pl.pallas_call(
        _dist_body,
        grid=(GM, GN),
        in_specs=[
            pl.BlockSpec((MB, E), lambda m, n: (m, 0)),
            pl.BlockSpec((NB, E), lambda m, n: (n, 0)),
            pl.BlockSpec((MB, 1), lambda m, n: (m, 0)),
            pl.BlockSpec((1, NB), lambda m, n: (0, n)),
        ],
        out_specs=[
            pl.BlockSpec((MB,), lambda m, n: (m,)),
            pl.BlockSpec(memory_space=pltpu.SMEM, block_shape=(1,),
                         index_map=lambda m, n: (0,)),
        ],
        out_shape=[
            jax.ShapeDtypeStruct((K_TOK,), jnp.int32),
            jax.ShapeDtypeStruct((1,), jnp.float32),
        ],
        scratch_shapes=[
            pltpu.VMEM((MB, 1), jnp.float32),
            pltpu.VMEM((MB, 1), jnp.int32),
            pltpu.VMEM((MB, 1), jnp.float32),
        ],
    )(z_flat, W, zsq, wsq)


def _sc_gather(table, idx):
    try:
        info = plsc.get_sparse_core_info()
        nc, ns = info.num_cores, info.num_subcores
    except Exception:
        nc, ns = 2, 16
    nw = nc * ns
    b_per_w = K_TOK // nw
    ch = 128
    n_ch = b_per_w // ch
    mesh = plsc.VectorSubcoreMesh(core_axis_name="c", subcore_axis_name="s")

    @functools.partial(
        pl.kernel, mesh=mesh,
        out_type=jax.ShapeDtypeStruct((K_TOK, E), jnp.float32),
        scratch_types=[
            pltpu.VMEM((ch,), jnp.int32),
            pltpu.VMEM((ch, E), jnp.float32),
            pltpu.SemaphoreType.DMA,
        ],
    )
    def _gather(table_hbm, idx_hbm, out_hbm, idx_v, rows_v, sem):
        wid = lax.axis_index("s") * nc + lax.axis_index("c")
        base = wid * b_per_w
        for j in range(n_ch):
            off = base + j * ch
            pltpu.sync_copy(idx_hbm.at[pl.ds(off, ch)], idx_v)
            pltpu.async_copy(table_hbm.at[idx_v], rows_v, sem).wait()
            pltpu.sync_copy(rows_v, out_hbm.at[pl.ds(off, ch)])

    return _gather(table, idx)


def kernel(z, W):
    zp = jnp.transpose(z, (0, 2, 3, 1))
    z_flat = zp.reshape(K_TOK, E)
    zsq = jnp.sum(zp ** 2, axis=3).reshape(K_TOK, 1)
    wsq = jnp.sum(W ** 2, axis=1).reshape(1, K_CODE)
    idx, loss_sum = _distance_argmin(z_flat * 2.0, W, zsq, wsq)
    table = W.astype(jnp.bfloat16).astype(jnp.float32)
    vq_rows = _sc_gather(table, idx)
    loss = loss_sum[0] * jnp.float32(1.25 / N_ELEMS)
    vq_tok = z_flat + (vq_rows - z_flat)
    vq_out = jnp.transpose(vq_tok.reshape(8, 32, 32, E), (0, 3, 1, 2))
    return (loss, vq_out)
